# Initial kernel scaffold; baseline (speedup 1.0000x reference)
import jax
import jax.numpy as jnp
from jax.experimental import pallas as pl


def kernel(table, nodes, node2hyp):
    raise NotImplementedError("write your pallas kernel here")


# speedup vs baseline: 3.3791x; 3.3791x over previous
import functools

import jax
import jax.numpy as jnp
from jax import lax
from jax.experimental import pallas as pl
from jax.experimental.pallas import tpu as pltpu
from jax.experimental.pallas import tpu_sc as plsc

L = 16


def _mean_agg_body(consts, table_hbm, nodes_hbm, n2h_hbm, out_hbm,
                   nid_ref, hyp_ref, tbuf_ref, midxa, midxb,
                   rref, rows_ref, accv_ref, sem):
    (B, DEG, D, SENT, NT, NC, NS, TPW) = consts
    NW = NC * NS
    DC = D // L
    TN = 8
    ROWS = TN * DEG
    NIDC = TPW * TN // 4

    cid = lax.axis_index("c")
    sid = lax.axis_index("s")
    wid = sid * NC + cid

    start_t = (wid * NT) // NW
    end_t = ((wid + 1) * NT) // NW
    my_tiles = end_t - start_t
    node0 = start_t * TN

    iota = lax.iota(jnp.int32, L)
    mlow = iota < (TN // 2)
    mhigh = (iota >= (TN // 2)) & (iota < TN)

    for j in range(4):
        pltpu.sync_copy(nodes_hbm.at[pl.ds(node0 + j * NIDC, NIDC)],
                        nid_ref.at[j])
    for j in range(4):
        pltpu.sync_copy(n2h_hbm.at[nid_ref.at[j]],
                        hyp_ref.at[pl.ds(j * NIDC, NIDC)])

    def tile_body(t, carry):
        rowbase = t * TN
        for b in range(TN):
            hrow = hyp_ref.at[rowbase + b]
            for h in range(DEG // L):
                v = hrow[pl.ds(h * L, L)]
                plsc.store_scatter(tbuf_ref, [(iota + h * L) * L + b], v)

        idv = [tbuf_ref[pl.ds(j * L, L)] for j in range(DEG)]

        dupcnt = jnp.zeros((L,), jnp.int32)
        sent = jnp.full((L,), SENT, jnp.int32)
        for j in range(DEG):
            if j == 0:
                mval = idv[0]
            else:
                d = idv[j] == idv[0]
                for k in range(1, j):
                    d = d | (idv[j] == idv[k])
                dupcnt = dupcnt + jnp.where(d, 1, 0)
                mval = jnp.where(d, sent, idv[j])
            flat = iota * DEG + j
            plsc.store_scatter(midxa, [flat], mval, mask=mlow)
            plsc.store_scatter(midxb, [flat - 128], mval, mask=mhigh)

        cnt = jnp.full((L,), DEG, jnp.int32) - dupcnt
        rref[...] = 1.0 / cnt.astype(jnp.float32)

        cp0 = pltpu.async_copy(table_hbm.at[midxa],
                               rows_ref.at[pl.ds(0, 128)], sem)
        cp1 = pltpu.async_copy(table_hbm.at[midxb],
                               rows_ref.at[pl.ds(128, 128)], sem)
        cp0.wait()
        cp1.wait()

        for b in range(TN):
            rb = plsc.load_gather(rref, [jnp.full((L,), b, jnp.int32)])
            first = rows_ref.at[b * DEG]
            init = tuple(first[pl.ds(c * L, L)] for c in range(DC))

            def jbody(j, accs, _b=b):
                rr = rows_ref.at[_b * DEG + j]
                return tuple(accs[c] + rr[pl.ds(c * L, L)]
                             for c in range(DC))

            accs = lax.fori_loop(1, DEG, jbody, init)
            orow = accv_ref.at[b]
            for c in range(DC):
                orow[pl.ds(c * L, L)] = accs[c] * rb
        pltpu.sync_copy(accv_ref,
                        out_hbm.at[pl.ds(node0 + rowbase, TN)])
        return carry

    lax.fori_loop(0, my_tiles, tile_body, 0)


def kernel(table, nodes, node2hyp):
    V, D = table.shape
    B = nodes.shape[0]
    DEG = node2hyp.shape[1]
    SENT = V

    info = plsc.get_sparse_core_info()
    NC, NS = info.num_cores, info.num_subcores
    NW = NC * NS
    TN = 8
    NT = B // TN
    TPW = -(-NT // NW)

    table_aug = jnp.concatenate(
        [table, jnp.zeros((16, D), table.dtype)], axis=0)

    mesh = plsc.VectorSubcoreMesh(core_axis_name="c", subcore_axis_name="s")
    body = functools.partial(
        _mean_agg_body, (B, DEG, D, SENT, NT, NC, NS, TPW))

    f = pl.kernel(
        body,
        out_type=jax.ShapeDtypeStruct((B, D), jnp.float32),
        mesh=mesh,
        compiler_params=pltpu.CompilerParams(
            needs_layout_passes=False, use_tc_tiling_on_sc=False),
        scratch_types=[
            pltpu.VMEM((4, TPW * TN // 4), jnp.int32),
            pltpu.VMEM((TPW * TN, DEG), jnp.int32),
            pltpu.VMEM((DEG * L,), jnp.int32),
            pltpu.VMEM((128,), jnp.int32),
            pltpu.VMEM((128,), jnp.int32),
            pltpu.VMEM((L,), jnp.float32),
            pltpu.VMEM((TN * DEG, D), jnp.float32),
            pltpu.VMEM((TN, D), jnp.float32),
            pltpu.SemaphoreType.DMA,
        ],
    )
    return f(table_aug, nodes, node2hyp)


# speedup vs baseline: 4.7573x; 1.4079x over previous
import functools

import jax
import jax.numpy as jnp
from jax import lax
from jax.experimental import pallas as pl
from jax.experimental.pallas import tpu as pltpu
from jax.experimental.pallas import tpu_sc as plsc

L = 16


def _mean_agg_body(consts, table_hbm, nodes_hbm, n2h_hbm, out_hbm,
                   nid_ref, hyp_ref, tbuf_ref, midx_ref,
                   rref, rows_ref, accv_ref, sem):
    (B, DEG, D, SENT, NT, NC, NS, TPW) = consts
    NW = NC * NS
    DC = D // L
    TN = 8
    ROWS = TN * DEG
    NIDC = TPW * TN // 4

    cid = lax.axis_index("c")
    sid = lax.axis_index("s")
    wid = sid * NC + cid

    start_t = (wid * NT) // NW
    end_t = ((wid + 1) * NT) // NW
    my_tiles = end_t - start_t
    node0 = start_t * TN

    iota = lax.iota(jnp.int32, L)
    mlow = iota < (TN // 2)
    mhigh = (iota >= (TN // 2)) & (iota < TN)

    for j in range(4):
        pltpu.sync_copy(nodes_hbm.at[pl.ds(node0 + j * NIDC, NIDC)],
                        nid_ref.at[j])
    for j in range(4):
        pltpu.sync_copy(n2h_hbm.at[nid_ref.at[j]],
                        hyp_ref.at[pl.ds(j * NIDC, NIDC)])

    def gather_descs(par, tile_in_buf_unused=None):
        src0 = table_hbm.at[midx_ref.at[2 * par]]
        src1 = table_hbm.at[midx_ref.at[2 * par + 1]]
        dst0 = rows_ref.at[pl.ds(par * ROWS, 128)]
        dst1 = rows_ref.at[pl.ds(par * ROWS + 128, 128)]
        return (src0, dst0), (src1, dst1)

    def prepare(t, par):
        rowbase = t * TN
        for b in range(TN):
            hrow = hyp_ref.at[rowbase + b]
            for h in range(DEG // L):
                v = hrow[pl.ds(h * L, L)]
                plsc.store_scatter(tbuf_ref, [(iota + h * L) * L + b], v)

        idv = [tbuf_ref[pl.ds(j * L, L)] for j in range(DEG)]

        dupcnt = jnp.zeros((L,), jnp.int32)
        sent = jnp.full((L,), SENT, jnp.int32)
        ma = midx_ref.at[2 * par]
        mb = midx_ref.at[2 * par + 1]
        for j in range(DEG):
            if j == 0:
                mval = idv[0]
            else:
                d = idv[j] == idv[0]
                for k in range(1, j):
                    d = d | (idv[j] == idv[k])
                dupcnt = dupcnt + jnp.where(d, 1, 0)
                mval = jnp.where(d, sent, idv[j])
            flat = iota * DEG + j
            plsc.store_scatter(ma, [flat], mval, mask=mlow)
            plsc.store_scatter(mb, [flat - 128], mval, mask=mhigh)

        cnt = jnp.full((L,), DEG, jnp.int32) - dupcnt
        rref[pl.ds(par * L, L)] = 1.0 / cnt.astype(jnp.float32)

    def reduce_store(t, par):
        base = par * ROWS
        for b in range(TN):
            rb = plsc.load_gather(
                rref, [jnp.full((L,), b, jnp.int32) + par * L])
            zero = jnp.zeros((L,), jnp.float32)
            init = (zero,) * DC

            def jbody(k, accs, _b=b):
                r0 = rows_ref.at[base + _b * DEG + 4 * k]
                r1 = rows_ref.at[base + _b * DEG + 4 * k + 1]
                r2 = rows_ref.at[base + _b * DEG + 4 * k + 2]
                r3 = rows_ref.at[base + _b * DEG + 4 * k + 3]
                return tuple(
                    accs[c]
                    + ((r0[pl.ds(c * L, L)] + r1[pl.ds(c * L, L)])
                       + (r2[pl.ds(c * L, L)] + r3[pl.ds(c * L, L)]))
                    for c in range(DC))

            accs = lax.fori_loop(0, DEG // 4, jbody, init)
            orow = accv_ref.at[b]
            for c in range(DC):
                orow[pl.ds(c * L, L)] = accs[c] * rb
        pltpu.sync_copy(accv_ref, out_hbm.at[pl.ds(node0 + t * TN, TN)])

    def pipe_body(i, carry):
        par = lax.rem(i, 2)
        opar = 1 - par

        @pl.when(i < my_tiles)
        def _fire():
            prepare(i, par)
            (s0, d0), (s1, d1) = gather_descs(par)
            pltpu.async_copy(s0, d0, sem)
            pltpu.async_copy(s1, d1, sem)

        @pl.when(i > 0)
        def _drain():
            (s0, d0), (s1, d1) = gather_descs(opar)
            pltpu.make_async_copy(s0, d0, sem).wait()
            pltpu.make_async_copy(s1, d1, sem).wait()
            reduce_store(i - 1, opar)

        return carry

    lax.fori_loop(0, my_tiles + 1, pipe_body, 0)


def kernel(table, nodes, node2hyp):
    V, D = table.shape
    B = nodes.shape[0]
    DEG = node2hyp.shape[1]
    SENT = V

    info = plsc.get_sparse_core_info()
    NC, NS = info.num_cores, info.num_subcores
    NW = NC * NS
    TN = 8
    NT = B // TN
    TPW = -(-NT // NW)

    table_aug = jnp.concatenate(
        [table, jnp.zeros((16, D), table.dtype)], axis=0)

    mesh = plsc.VectorSubcoreMesh(core_axis_name="c", subcore_axis_name="s")
    body = functools.partial(
        _mean_agg_body, (B, DEG, D, SENT, NT, NC, NS, TPW))

    f = pl.kernel(
        body,
        out_type=jax.ShapeDtypeStruct((B, D), jnp.float32),
        mesh=mesh,
        compiler_params=pltpu.CompilerParams(
            needs_layout_passes=False, use_tc_tiling_on_sc=False),
        scratch_types=[
            pltpu.VMEM((4, TPW * TN // 4), jnp.int32),
            pltpu.VMEM((TPW * TN, DEG), jnp.int32),
            pltpu.VMEM((DEG * L,), jnp.int32),
            pltpu.VMEM((4, 128), jnp.int32),
            pltpu.VMEM((2 * L,), jnp.float32),
            pltpu.VMEM((2 * TN * DEG, D), jnp.float32),
            pltpu.VMEM((TN, D), jnp.float32),
            pltpu.SemaphoreType.DMA,
        ],
    )
    return f(table_aug, nodes, node2hyp)
